# baseline (speedup 1.0000x reference)
import functools

import jax
import jax.numpy as jnp
from jax import lax
from jax.experimental import pallas as pl
from jax.experimental.pallas import tpu as pltpu
from jax.experimental.pallas import tpu_sc as plsc

B, L, K, H = 512, 50, 8, 1024
NE, NR = 20000, 500
ROWS = L * B
NC, NS = 2, 16
NW = NC * NS
RPW = ROWS // NW
G = 4
NG = RPW // G
NBUF = 4
ENT_PW = B // NW
H2 = H // 2
LANES = 16
PACK_BLK = 1000


def _tc_pack_table(x_ref, out_ref):
    def bits(v):
        return lax.bitcast_convert_type(
            v.astype(jnp.bfloat16).astype(jnp.float32), jnp.uint32)

    x = x_ref[...]
    word = bits(x[:, H2:]) | (bits(x[:, :H2]) >> 16)
    out_ref[...] = lax.bitcast_convert_type(word, jnp.int32)


def _sc_gather_sum(idx_hbm, eids_hbm, table_pk_hbm, table_hbm, sums_out,
                   ent_out, idx_v, gbuf, sbuf, eidx_v, ebuf,
                   gsems, wsems, esem):
    wid = lax.axis_index("s") * NC + lax.axis_index("c")
    base = wid * RPW

    pltpu.sync_copy(idx_hbm.at[pl.ds(base * K, RPW * K)], idx_v)

    pltpu.sync_copy(eids_hbm.at[pl.ds(wid * ENT_PW, ENT_PW)], eidx_v)
    pltpu.async_copy(table_hbm.at[eidx_v], ebuf, esem).wait()
    pltpu.sync_copy(ebuf, ent_out.at[pl.ds(wid * ENT_PW, ENT_PW)])

    def start_gather(slot, g):
        pltpu.make_async_copy(
            table_pk_hbm.at[idx_v.at[pl.ds(pl.multiple_of(g * (G * K), G * K),
                                           G * K)]],
            gbuf.at[slot], gsems[slot]).start()

    def wait_gather(slot):
        pltpu.make_async_copy(
            table_pk_hbm.at[idx_v.at[pl.ds(0, G * K)]],
            gbuf.at[slot], gsems[slot]).wait()

    def start_write(slot, g):
        pltpu.make_async_copy(
            sbuf.at[slot], sums_out.at[pl.ds(base + g * G, G)],
            wsems[slot]).start()

    def wait_write(slot):
        pltpu.make_async_copy(
            sbuf.at[slot], sums_out.at[pl.ds(base, G)], wsems[slot]).wait()

    def sum_group(slot):
        gb = gbuf.at[slot].bitcast(jnp.bfloat16)
        sb = sbuf.at[slot].bitcast(jnp.bfloat16)

        def jbody(j, carry):
            off = pl.multiple_of(j * LANES, LANES)
            for r in range(G):
                v = [gb[pl.ds(2 * (r * K + k), 2), pl.ds(off, LANES)]
                     for k in range(K)]
                s = ((v[0] + v[1]) + (v[2] + v[3])) + ((v[4] + v[5]) + (v[6] + v[7]))
                sb[pl.ds(2 * r, 2), pl.ds(off, LANES)] = s
            return carry

        lax.fori_loop(0, H2 // LANES, jbody, 0, unroll=2)

    for s in range(NBUF):
        start_gather(s, s)

    def body(i, carry):
        for s in range(NBUF):
            g = NBUF * i + s
            wait_gather(s)

            @pl.when(i > 0)
            def _():
                wait_write(s)

            sum_group(s)

            start_write(s, g)

            @pl.when(g + NBUF < NG)
            def _():
                start_gather(s, g + NBUF)
        return carry

    lax.fori_loop(0, NG // NBUF, body, 0)
    for s in range(NBUF):
        wait_write(s)


@functools.partial(
    pl.kernel,
    out_type=(jax.ShapeDtypeStruct((ROWS, H2), jnp.int32),
              jax.ShapeDtypeStruct((B, H), jnp.float32)),
    mesh=plsc.VectorSubcoreMesh(core_axis_name="c", subcore_axis_name="s"),
    scratch_types=[
        pltpu.VMEM((RPW * K,), jnp.int32),
        pltpu.VMEM((NBUF, G * K, H2), jnp.int32),
        pltpu.VMEM((NBUF, G, H2), jnp.int32),
        pltpu.VMEM((ENT_PW,), jnp.int32),
        pltpu.VMEM((ENT_PW, H), jnp.float32),
        [pltpu.SemaphoreType.DMA] * NBUF,
        [pltpu.SemaphoreType.DMA] * NBUF,
        pltpu.SemaphoreType.DMA,
    ],
)
def _sc_stage(idx_hbm, eids_hbm, table_pk_hbm, table_hbm, sums_out, ent_out,
              *scratch):
    _sc_gather_sum(idx_hbm, eids_hbm, table_pk_hbm, table_hbm, sums_out,
                   ent_out, *scratch)


def _tc_assemble(rel_ref, ent_ref, sums_ref, w_ref, b_ref, out_ref):
    j = pl.program_id(1)

    @pl.when(j == 0)
    def _():
        out_ref[...] = ent_ref[...]

    @pl.when(j == 1)
    def _():
        rel_mean = jnp.sum(rel_ref[...], axis=0, keepdims=True) * (1.0 / NR)
        out_ref[...] = jnp.broadcast_to(rel_mean, (B, H))

    @pl.when(j == 2)
    def _():
        w = lax.bitcast_convert_type(sums_ref[...], jnp.uint32)
        lo = lax.bitcast_convert_type(w << 16, jnp.float32)
        hi = lax.bitcast_convert_type(w & jnp.uint32(0xFFFF0000), jnp.float32)
        p0 = lax.dot_general(lo.astype(jnp.bfloat16), w_ref[:, :H2],
                             (((1,), (1,)), ((), ())),
                             preferred_element_type=jnp.float32)
        p1 = lax.dot_general(hi.astype(jnp.bfloat16), w_ref[:, H2:],
                             (((1,), (1,)), ((), ())),
                             preferred_element_type=jnp.float32)
        out_ref[...] = p0 + p1 + b_ref[...]


def kernel(entity_ids, neighbor_ids, history_times, entity_embeds,
           rel_embeds, W, b):
    del history_times
    idx_tm = jnp.transpose(neighbor_ids, (1, 0, 2)).reshape(ROWS * K)
    idx_tm = idx_tm.astype(jnp.int32)
    w_scaled = (W * (1.0 / K)).astype(jnp.bfloat16)

    table_pk = pl.pallas_call(
        _tc_pack_table,
        grid=(NE // PACK_BLK,),
        in_specs=[pl.BlockSpec((PACK_BLK, H), lambda i: (i, 0))],
        out_specs=pl.BlockSpec((PACK_BLK, H2), lambda i: (i, 0)),
        out_shape=jax.ShapeDtypeStruct((NE, H2), jnp.int32),
    )(entity_embeds)

    sums_pk, ent_rows = _sc_stage(idx_tm, entity_ids.astype(jnp.int32),
                                  table_pk, entity_embeds)

    packed = pl.pallas_call(
        _tc_assemble,
        grid=(L, 3),
        in_specs=[
            pl.BlockSpec((NR, H), lambda i, j: (0, 0)),
            pl.BlockSpec((B, H), lambda i, j: (0, 0)),
            pl.BlockSpec((B, H2), lambda i, j: (i, 0)),
            pl.BlockSpec((H, H), lambda i, j: (0, 0)),
            pl.BlockSpec((1, H), lambda i, j: (0, 0)),
        ],
        out_specs=pl.BlockSpec((B, H), lambda i, j: (i, j)),
        out_shape=jax.ShapeDtypeStruct((ROWS, 3 * H), jnp.float32),
    )(rel_embeds, ent_rows, sums_pk, w_scaled, b.reshape(1, H))

    hist_lengths = jnp.full((B,), L, dtype=jnp.int32)
    return (packed, hist_lengths)


# speedup vs baseline: 1.0743x; 1.0743x over previous
import functools

import jax
import jax.numpy as jnp
from jax import lax
from jax.experimental import pallas as pl
from jax.experimental.pallas import tpu as pltpu
from jax.experimental.pallas import tpu_sc as plsc

B, L, K, H = 512, 50, 8, 1024
NE, NR = 20000, 500
ROWS = L * B
NC, NS = 2, 16
NW = NC * NS
RPW = ROWS // NW
G = 4
NG = RPW // G
NBUF = 5
ENT_PW = B // NW
H2 = H // 2
LANES = 16
PACK_BLK = 1000


def _tc_pack_table(x_ref, out_ref):
    def bits(v):
        return lax.bitcast_convert_type(
            v.astype(jnp.bfloat16).astype(jnp.float32), jnp.uint32)

    x = x_ref[...]
    word = bits(x[:, H2:]) | (bits(x[:, :H2]) >> 16)
    out_ref[...] = lax.bitcast_convert_type(word, jnp.int32)


def _sc_gather_sum(idx_hbm, eids_hbm, table_pk_hbm, table_hbm, sums_out,
                   ent_out, idx_v, gbuf, sbuf, eidx_v, ebuf,
                   gsems, wsems, esem):
    wid = lax.axis_index("s") * NC + lax.axis_index("c")
    base = wid * RPW

    pltpu.sync_copy(idx_hbm.at[pl.ds(base * K, RPW * K)], idx_v)

    pltpu.sync_copy(eids_hbm.at[pl.ds(wid * ENT_PW, ENT_PW)], eidx_v)
    pltpu.async_copy(table_hbm.at[eidx_v], ebuf, esem).wait()
    pltpu.sync_copy(ebuf, ent_out.at[pl.ds(wid * ENT_PW, ENT_PW)])

    def start_gather(slot, g):
        pltpu.make_async_copy(
            table_pk_hbm.at[idx_v.at[pl.ds(pl.multiple_of(g * (G * K), G * K),
                                           G * K)]],
            gbuf.at[slot], gsems[slot]).start()

    def wait_gather(slot):
        pltpu.make_async_copy(
            table_pk_hbm.at[idx_v.at[pl.ds(0, G * K)]],
            gbuf.at[slot], gsems[slot]).wait()

    def start_write(slot, g):
        pltpu.make_async_copy(
            sbuf.at[slot], sums_out.at[pl.ds(base + g * G, G)],
            wsems[slot]).start()

    def wait_write(slot):
        pltpu.make_async_copy(
            sbuf.at[slot], sums_out.at[pl.ds(base, G)], wsems[slot]).wait()

    def sum_group(slot):
        gb = gbuf.at[slot].bitcast(jnp.bfloat16)
        sb = sbuf.at[slot].bitcast(jnp.bfloat16)

        def jbody(j, carry):
            off = pl.multiple_of(j * LANES, LANES)
            for r in range(G):
                v = [gb[pl.ds(2 * (r * K + k), 2), pl.ds(off, LANES)]
                     for k in range(K)]
                s = ((v[0] + v[1]) + (v[2] + v[3])) + ((v[4] + v[5]) + (v[6] + v[7]))
                sb[pl.ds(2 * r, 2), pl.ds(off, LANES)] = s
            return carry

        lax.fori_loop(0, H2 // LANES, jbody, 0, unroll=4)

    for s in range(NBUF):
        start_gather(s, s)

    def body(i, carry):
        for s in range(NBUF):
            g = NBUF * i + s
            wait_gather(s)

            @pl.when(i > 0)
            def _():
                wait_write(s)

            sum_group(s)

            start_write(s, g)

            @pl.when(g + NBUF < NG)
            def _():
                start_gather(s, g + NBUF)
        return carry

    lax.fori_loop(0, NG // NBUF, body, 0)
    for s in range(NBUF):
        wait_write(s)


@functools.partial(
    pl.kernel,
    out_type=(jax.ShapeDtypeStruct((ROWS, H2), jnp.int32),
              jax.ShapeDtypeStruct((B, H), jnp.float32)),
    mesh=plsc.VectorSubcoreMesh(core_axis_name="c", subcore_axis_name="s"),
    scratch_types=[
        pltpu.VMEM((RPW * K,), jnp.int32),
        pltpu.VMEM((NBUF, G * K, H2), jnp.int32),
        pltpu.VMEM((NBUF, G, H2), jnp.int32),
        pltpu.VMEM((ENT_PW,), jnp.int32),
        pltpu.VMEM((ENT_PW, H), jnp.float32),
        [pltpu.SemaphoreType.DMA] * NBUF,
        [pltpu.SemaphoreType.DMA] * NBUF,
        pltpu.SemaphoreType.DMA,
    ],
)
def _sc_stage(idx_hbm, eids_hbm, table_pk_hbm, table_hbm, sums_out, ent_out,
              *scratch):
    _sc_gather_sum(idx_hbm, eids_hbm, table_pk_hbm, table_hbm, sums_out,
                   ent_out, *scratch)


def _tc_rel_fill(rel_ref, out_ref):
    rel_mean = jnp.sum(rel_ref[...], axis=0, keepdims=True) * (1.0 / NR)
    out_ref[...] = jnp.broadcast_to(rel_mean, (B, H))


def _tc_ent_proj(ent_ref, sums_ref, w_ref, b_ref, aliased_ref, out_ref):
    del aliased_ref
    j = pl.program_id(1)

    @pl.when(j == 0)
    def _():
        out_ref[...] = ent_ref[...]

    @pl.when(j == 1)
    def _():
        w = lax.bitcast_convert_type(sums_ref[...], jnp.uint32)
        lo = lax.bitcast_convert_type(w << 16, jnp.float32)
        hi = lax.bitcast_convert_type(w & jnp.uint32(0xFFFF0000), jnp.float32)
        p0 = lax.dot_general(lo.astype(jnp.bfloat16), w_ref[:, :H2],
                             (((1,), (1,)), ((), ())),
                             preferred_element_type=jnp.float32)
        p1 = lax.dot_general(hi.astype(jnp.bfloat16), w_ref[:, H2:],
                             (((1,), (1,)), ((), ())),
                             preferred_element_type=jnp.float32)
        out_ref[...] = p0 + p1 + b_ref[...]


def kernel(entity_ids, neighbor_ids, history_times, entity_embeds,
           rel_embeds, W, b):
    del history_times
    idx_tm = jnp.transpose(neighbor_ids, (1, 0, 2)).reshape(ROWS * K)
    idx_tm = idx_tm.astype(jnp.int32)
    w_scaled = (W * (1.0 / K)).astype(jnp.bfloat16)

    table_pk = pl.pallas_call(
        _tc_pack_table,
        grid=(NE // PACK_BLK,),
        in_specs=[pl.BlockSpec((PACK_BLK, H), lambda i: (i, 0))],
        out_specs=pl.BlockSpec((PACK_BLK, H2), lambda i: (i, 0)),
        out_shape=jax.ShapeDtypeStruct((NE, H2), jnp.int32),
    )(entity_embeds)

    sums_pk, ent_rows = _sc_stage(idx_tm, entity_ids.astype(jnp.int32),
                                  table_pk, entity_embeds)

    packed0 = pl.pallas_call(
        _tc_rel_fill,
        grid=(L,),
        in_specs=[pl.BlockSpec((NR, H), lambda i: (0, 0))],
        out_specs=pl.BlockSpec((B, H), lambda i: (i, 1)),
        out_shape=jax.ShapeDtypeStruct((ROWS, 3 * H), jnp.float32),
    )(rel_embeds)

    packed = pl.pallas_call(
        _tc_ent_proj,
        grid=(L, 2),
        in_specs=[
            pl.BlockSpec((B, H), lambda i, j: (0, 0)),
            pl.BlockSpec((B, H2), lambda i, j: (i, 0)),
            pl.BlockSpec((H, H), lambda i, j: (0, 0)),
            pl.BlockSpec((1, H), lambda i, j: (0, 0)),
            pl.BlockSpec(memory_space=pltpu.MemorySpace.HBM),
        ],
        out_specs=pl.BlockSpec((B, H), lambda i, j: (i, 2 * j)),
        out_shape=jax.ShapeDtypeStruct((ROWS, 3 * H), jnp.float32),
        input_output_aliases={4: 0},
    )(ent_rows, sums_pk, w_scaled, b.reshape(1, H), packed0)

    hist_lengths = jnp.full((B,), L, dtype=jnp.int32)
    return (packed, hist_lengths)
